# baseline (speedup 1.0000x reference)
import functools

import jax
import jax.numpy as jnp
import numpy as np
from jax.experimental import pallas as pl
from jax.experimental.pallas import tpu as pltpu

_IMG_H = 512
_IMG_W = 512
_C_OUT = 128
_NY = 200
_NX = 200
_DB = 64
_DEPTH_MIN = 1.0
_DEPTH_MAX = 60.0
_SIGMA = 0.8
_MIN_OP = 0.05
_EPS = 1e-6
_PC = np.array([-51.2, -51.2, -20.0, 51.2, 51.2, 20.0], dtype=np.float32)
_VS = np.array([0.512, 0.512, 40.0], dtype=np.float32)
_PAD = 2
_NYP = _NY + 2 * _PAD
_NXP = _NX + 2 * _PAD
_TILES = 8
_ROWS = _NY // _TILES


def _conv(x, w, stride, pad):
    return jax.lax.conv_general_dilated(
        x, w, (stride, stride), [(pad, pad), (pad, pad)],
        dimension_numbers=('NCHW', 'OIHW', 'NCHW'))


def _bn(x, g, b, m, v):
    return (x - m[None, :, None, None]) / jnp.sqrt(v[None, :, None, None] + 1e-5) * g[None, :, None, None] + b[None, :, None, None]


def _mm_bn_relu_kernel(x_ref, w_ref, s_ref, t_ref, o_ref):
    acc = jnp.dot(x_ref[...], w_ref[...], preferred_element_type=jnp.float32,
                  precision=jax.lax.Precision.HIGHEST)
    o_ref[...] = jnp.maximum(acc * s_ref[...] + t_ref[...], 0.0)


def _mm_bn_relu(x, w, s, t, bm):
    m, k = x.shape
    n = w.shape[1]
    return pl.pallas_call(
        _mm_bn_relu_kernel,
        grid=(m // bm,),
        in_specs=[
            pl.BlockSpec((bm, k), lambda i: (i, 0)),
            pl.BlockSpec((k, n), lambda i: (0, 0)),
            pl.BlockSpec((1, n), lambda i: (0, 0)),
            pl.BlockSpec((1, n), lambda i: (0, 0)),
        ],
        out_specs=pl.BlockSpec((bm, n), lambda i: (i, 0)),
        out_shape=jax.ShapeDtypeStruct((m, n), jnp.float32),
        compiler_params=pltpu.CompilerParams(dimension_semantics=("parallel",)),
    )(x, w, s, t)


def _proj_scale_kernel(x_ref, w_ref, b_ref, wv_ref, o_ref):
    acc = jnp.dot(x_ref[...], w_ref[...], preferred_element_type=jnp.float32,
                  precision=jax.lax.Precision.HIGHEST)
    o_ref[...] = (acc + b_ref[...]) * wv_ref[...]


def _proj_scale(x, w, b, wv):
    m, k = x.shape
    n = w.shape[1]
    return pl.pallas_call(
        _proj_scale_kernel,
        out_shape=jax.ShapeDtypeStruct((m, n), jnp.float32),
    )(x, w, b, wv)


def _scatter_kernel(py_ref, px_ref, wv_ref, pf_ref, out_ref, canvas_ref, wacc_ref):
    t = pl.program_id(1)
    dyg = jax.lax.broadcasted_iota(jnp.int32, (3, 3), 0).astype(jnp.float32) - 1.0
    dxg = jax.lax.broadcasted_iota(jnp.int32, (3, 3), 1).astype(jnp.float32) - 1.0
    kw = jnp.exp(-(dxg * dxg + dyg * dyg) / (2.0 * _SIGMA * _SIGMA))

    @pl.when(t == 0)
    def _():
        canvas_ref[...] = jnp.zeros(canvas_ref.shape, canvas_ref.dtype)
        wacc_ref[...] = jnp.zeros(wacc_ref.shape, wacc_ref.dtype)

        def body(j, carry):
            w = wv_ref[0, 0, j]
            py = py_ref[0, 0, j]
            px = px_ref[0, 0, j]
            f = pf_ref[0, j, :]
            kww = kw * w
            patch = canvas_ref[pl.ds(py, 3), pl.ds(px, 3), :]
            canvas_ref[pl.ds(py, 3), pl.ds(px, 3), :] = patch + kw[:, :, None] * f[None, None, :]
            wp = wacc_ref[pl.ds(py, 3), pl.ds(px, 3), :]
            wacc_ref[pl.ds(py, 3), pl.ds(px, 3), :] = wp + kww[:, :, None]
            return carry

        jax.lax.fori_loop(0, py_ref.shape[2], body, 0)

    y0 = t * _ROWS + _PAD
    c = canvas_ref[pl.ds(y0, _ROWS), _PAD:_PAD + _NX, :]
    wv = wacc_ref[pl.ds(y0, _ROWS), _PAD:_PAD + _NX, 0:1]
    out_ref[0] = c / jnp.maximum(wv, _EPS)


def _bev_scatter(py, px, wv, pf, B, N):
    return pl.pallas_call(
        _scatter_kernel,
        grid=(B, _TILES),
        in_specs=[
            pl.BlockSpec((1, 1, N), lambda b, t: (b, 0, 0), memory_space=pltpu.SMEM),
            pl.BlockSpec((1, 1, N), lambda b, t: (b, 0, 0), memory_space=pltpu.SMEM),
            pl.BlockSpec((1, 1, N), lambda b, t: (b, 0, 0), memory_space=pltpu.SMEM),
            pl.BlockSpec((1, N, _C_OUT), lambda b, t: (b, 0, 0)),
        ],
        out_specs=pl.BlockSpec((1, _ROWS, _NX, _C_OUT), lambda b, t: (b, t, 0, 0)),
        out_shape=jax.ShapeDtypeStruct((B, _NY, _NX, _C_OUT), jnp.float32),
        scratch_shapes=[
            pltpu.VMEM((_NYP, _NXP, _C_OUT), jnp.float32),
            pltpu.VMEM((_NYP, _NXP, 8), jnp.float32),
        ],
        compiler_params=pltpu.CompilerParams(
            dimension_semantics=("parallel", "arbitrary")),
    )(py.reshape(B, 1, N), px.reshape(B, 1, N), wv.reshape(B, 1, N), pf)


def kernel(images, camera_projection, t_lidar_camera, params):
    p = params
    x = images
    for i in range(1, 5):
        x = _conv(x, p['enc_w%d' % i], 2, 1)
        x = _bn(x, p['enc_g%d' % i], p['enc_b%d' % i], p['enc_m%d' % i], p['enc_v%d' % i])
        x = jnp.maximum(x, 0.0)
    feats = x
    depth_logits = _conv(feats, p['dh_w'], 1, 0) + p['dh_bias'][None, :, None, None]
    depth_probs = jax.nn.softmax(depth_logits, axis=1)
    opacity = jax.nn.sigmoid(_conv(feats, p['oh_w'], 1, 0) + p['oh_bias'][None, :, None, None])[:, 0]
    B, C, Hf, Wf = feats.shape
    depth_values = jnp.linspace(_DEPTH_MIN, _DEPTH_MAX, _DB).astype(jnp.float32)
    z = jnp.sum(depth_probs * depth_values[None, :, None, None], axis=1)[:, None]
    ys = (jnp.arange(Hf, dtype=jnp.float32) + 0.5) * (float(_IMG_H) / float(Hf))
    xs = (jnp.arange(Wf, dtype=jnp.float32) + 0.5) * (float(_IMG_W) / float(Wf))
    yy, xx = jnp.meshgrid(ys, xs, indexing='ij')
    yy = yy[None, None]
    xx = xx[None, None]
    fx = jnp.clip(camera_projection[:, 0, 0], _EPS, None).reshape(B, 1, 1, 1)
    fy = jnp.clip(camera_projection[:, 1, 1], _EPS, None).reshape(B, 1, 1, 1)
    cx = camera_projection[:, 0, 2].reshape(B, 1, 1, 1)
    cy = camera_projection[:, 1, 2].reshape(B, 1, 1, 1)
    x_cam = (xx - cx) * z / fx
    y_cam = (yy - cy) * z / fy
    cam_homo = jnp.stack([x_cam, y_cam, z, jnp.ones_like(z)], axis=-1).reshape(B, -1, 4)
    lidar = jnp.einsum('bij,bnj->bni', t_lidar_camera, cam_homo)[..., :3]
    pc = jnp.asarray(_PC)
    vs = jnp.asarray(_VS)
    x_i = jnp.floor((lidar[..., 0] - pc[0]) / vs[0]).astype(jnp.int32)
    y_i = jnp.floor((lidar[..., 1] - pc[1]) / vs[1]).astype(jnp.int32)
    z_ok = (lidar[..., 2] >= pc[2]) & (lidar[..., 2] < pc[5])
    N = Hf * Wf
    op_b = opacity.reshape(B, N)
    base_valid = (op_b >= _MIN_OP) & z_ok
    base_w = op_b * base_valid.astype(jnp.float32)
    ok = base_valid & (x_i >= -1) & (x_i <= _NX) & (y_i >= -1) & (y_i <= _NY)
    wv = jnp.where(ok, base_w, 0.0)
    px = jnp.where(ok, x_i + 1, 0).astype(jnp.int32)
    py = jnp.where(ok, y_i + 1, 0).astype(jnp.int32)

    fh_nhwc = feats.transpose(0, 2, 3, 1)
    xp = jnp.pad(fh_nhwc, ((0, 0), (1, 1), (1, 1), (0, 0)))
    cols = [xp[:, dy:dy + Hf, dx:dx + Wf, :] for dy in range(3) for dx in range(3)]
    x2d = jnp.concatenate(cols, axis=-1).reshape(B * N, 9 * C)
    w1 = p['fh_w1'].transpose(2, 3, 1, 0).reshape(9 * C, C)
    s1 = (p['fh_g1'] / jnp.sqrt(p['fh_v1'] + 1e-5)).reshape(1, C)
    t1 = (p['fh_b1'] - p['fh_m1'] * s1[0]).reshape(1, C)
    f1 = _mm_bn_relu(x2d, w1, s1, t1, 1024)
    w2 = p['fh_w2'][:, :, 0, 0].T
    b2 = p['fh_bias2'].reshape(1, C)
    pf = _proj_scale(f1, w2, b2, wv.reshape(B * N, 1))

    bev = _bev_scatter(py, px, wv, pf.reshape(B, N, C), B, N)
    return bev.transpose(0, 3, 1, 2)


# speedup vs baseline: 1.0480x; 1.0480x over previous
import functools

import jax
import jax.numpy as jnp
import numpy as np
from jax.experimental import pallas as pl
from jax.experimental.pallas import tpu as pltpu

_IMG_H = 512
_IMG_W = 512
_C_OUT = 128
_NY = 200
_NX = 200
_DB = 64
_DEPTH_MIN = 1.0
_DEPTH_MAX = 60.0
_SIGMA = 0.8
_MIN_OP = 0.05
_EPS = 1e-6
_PC = np.array([-51.2, -51.2, -20.0, 51.2, 51.2, 20.0], dtype=np.float32)
_VS = np.array([0.512, 0.512, 40.0], dtype=np.float32)
_PAD = 2
_NYP = _NY + 2 * _PAD
_NXP = _NX + 2 * _PAD
_TILES = 8
_ROWS = _NY // _TILES


def _conv(x, w, stride, pad):
    return jax.lax.conv_general_dilated(
        x, w, (stride, stride), [(pad, pad), (pad, pad)],
        dimension_numbers=('NCHW', 'OIHW', 'NCHW'))


def _bn(x, g, b, m, v):
    return (x - m[None, :, None, None]) / jnp.sqrt(v[None, :, None, None] + 1e-5) * g[None, :, None, None] + b[None, :, None, None]


def _fhead_kernel(xp_ref, w1_ref, s_ref, t_ref, w2_ref, b2_ref, wv_ref, o_ref):
    hf = xp_ref.shape[1] - 2
    wf = xp_ref.shape[2] - 2
    c = xp_ref.shape[3]
    acc = jnp.zeros((hf * wf, c), jnp.float32)
    k = 0
    for dy in range(3):
        for dx in range(3):
            xs = xp_ref[0, dy:dy + hf, dx:dx + wf, :].reshape(hf * wf, c)
            acc = acc + jnp.dot(xs, w1_ref[k], preferred_element_type=jnp.float32,
                                precision=jax.lax.Precision.HIGHEST)
            k += 1
    f1 = jnp.maximum(acc * s_ref[...] + t_ref[...], 0.0)
    pf = (jnp.dot(f1, w2_ref[...], preferred_element_type=jnp.float32,
                  precision=jax.lax.Precision.HIGHEST) + b2_ref[...]) * wv_ref[0]
    o_ref[0] = pf


def _feature_head(xp, w1, s, t, w2, b2, wv):
    b, hp, wp, c = xp.shape
    n = (hp - 2) * (wp - 2)
    return pl.pallas_call(
        _fhead_kernel,
        grid=(b,),
        in_specs=[
            pl.BlockSpec((1, hp, wp, c), lambda i: (i, 0, 0, 0)),
            pl.BlockSpec((9, c, c), lambda i: (0, 0, 0)),
            pl.BlockSpec((1, c), lambda i: (0, 0)),
            pl.BlockSpec((1, c), lambda i: (0, 0)),
            pl.BlockSpec((c, c), lambda i: (0, 0)),
            pl.BlockSpec((1, c), lambda i: (0, 0)),
            pl.BlockSpec((1, n, 1), lambda i: (i, 0, 0)),
        ],
        out_specs=pl.BlockSpec((1, n, c), lambda i: (i, 0, 0)),
        out_shape=jax.ShapeDtypeStruct((b, n, c), jnp.float32),
        compiler_params=pltpu.CompilerParams(dimension_semantics=("parallel",)),
    )(xp, w1, s, t, w2, b2, wv)


def _scatter_kernel(py_ref, px_ref, wv_ref, pf_ref, out_ref, canvas_ref, wacc_ref):
    t = pl.program_id(1)
    dyg = jax.lax.broadcasted_iota(jnp.int32, (3, 3), 0).astype(jnp.float32) - 1.0
    dxg = jax.lax.broadcasted_iota(jnp.int32, (3, 3), 1).astype(jnp.float32) - 1.0
    kw = jnp.exp(-(dxg * dxg + dyg * dyg) / (2.0 * _SIGMA * _SIGMA))

    @pl.when(t == 0)
    def _():
        canvas_ref[...] = jnp.zeros(canvas_ref.shape, canvas_ref.dtype)
        wacc_ref[...] = jnp.zeros(wacc_ref.shape, wacc_ref.dtype)

        def body(j, carry):
            w = wv_ref[0, 0, j]
            py = py_ref[0, 0, j]
            px = px_ref[0, 0, j]
            f = pf_ref[0, j, :]
            kww = kw * w
            patch = canvas_ref[pl.ds(py, 3), pl.ds(px, 3), :]
            canvas_ref[pl.ds(py, 3), pl.ds(px, 3), :] = patch + kw[:, :, None] * f[None, None, :]
            wp = wacc_ref[pl.ds(py, 3), pl.ds(px, 3), :]
            wacc_ref[pl.ds(py, 3), pl.ds(px, 3), :] = wp + kww[:, :, None]
            return carry

        jax.lax.fori_loop(0, py_ref.shape[2], body, 0)

    y0 = t * _ROWS + _PAD
    c = canvas_ref[pl.ds(y0, _ROWS), _PAD:_PAD + _NX, :]
    wv = wacc_ref[pl.ds(y0, _ROWS), _PAD:_PAD + _NX, 0:1]
    out_ref[0] = c / jnp.maximum(wv, _EPS)


def _bev_scatter(py, px, wv, pf, B, N):
    return pl.pallas_call(
        _scatter_kernel,
        grid=(B, _TILES),
        in_specs=[
            pl.BlockSpec((1, 1, N), lambda b, t: (b, 0, 0), memory_space=pltpu.SMEM),
            pl.BlockSpec((1, 1, N), lambda b, t: (b, 0, 0), memory_space=pltpu.SMEM),
            pl.BlockSpec((1, 1, N), lambda b, t: (b, 0, 0), memory_space=pltpu.SMEM),
            pl.BlockSpec((1, N, _C_OUT), lambda b, t: (b, 0, 0)),
        ],
        out_specs=pl.BlockSpec((1, _ROWS, _NX, _C_OUT), lambda b, t: (b, t, 0, 0)),
        out_shape=jax.ShapeDtypeStruct((B, _NY, _NX, _C_OUT), jnp.float32),
        scratch_shapes=[
            pltpu.VMEM((_NYP, _NXP, _C_OUT), jnp.float32),
            pltpu.VMEM((_NYP, _NXP, 8), jnp.float32),
        ],
        compiler_params=pltpu.CompilerParams(
            dimension_semantics=("parallel", "arbitrary")),
    )(py.reshape(B, 1, N), px.reshape(B, 1, N), wv.reshape(B, 1, N), pf)


def kernel(images, camera_projection, t_lidar_camera, params):
    p = params
    x = images
    for i in range(1, 5):
        x = _conv(x, p['enc_w%d' % i], 2, 1)
        x = _bn(x, p['enc_g%d' % i], p['enc_b%d' % i], p['enc_m%d' % i], p['enc_v%d' % i])
        x = jnp.maximum(x, 0.0)
    feats = x
    depth_logits = _conv(feats, p['dh_w'], 1, 0) + p['dh_bias'][None, :, None, None]
    depth_probs = jax.nn.softmax(depth_logits, axis=1)
    opacity = jax.nn.sigmoid(_conv(feats, p['oh_w'], 1, 0) + p['oh_bias'][None, :, None, None])[:, 0]
    B, C, Hf, Wf = feats.shape
    depth_values = jnp.linspace(_DEPTH_MIN, _DEPTH_MAX, _DB).astype(jnp.float32)
    z = jnp.sum(depth_probs * depth_values[None, :, None, None], axis=1)[:, None]
    ys = (jnp.arange(Hf, dtype=jnp.float32) + 0.5) * (float(_IMG_H) / float(Hf))
    xs = (jnp.arange(Wf, dtype=jnp.float32) + 0.5) * (float(_IMG_W) / float(Wf))
    yy, xx = jnp.meshgrid(ys, xs, indexing='ij')
    yy = yy[None, None]
    xx = xx[None, None]
    fx = jnp.clip(camera_projection[:, 0, 0], _EPS, None).reshape(B, 1, 1, 1)
    fy = jnp.clip(camera_projection[:, 1, 1], _EPS, None).reshape(B, 1, 1, 1)
    cx = camera_projection[:, 0, 2].reshape(B, 1, 1, 1)
    cy = camera_projection[:, 1, 2].reshape(B, 1, 1, 1)
    x_cam = (xx - cx) * z / fx
    y_cam = (yy - cy) * z / fy
    cam_homo = jnp.stack([x_cam, y_cam, z, jnp.ones_like(z)], axis=-1).reshape(B, -1, 4)
    lidar = jnp.einsum('bij,bnj->bni', t_lidar_camera, cam_homo)[..., :3]
    pc = jnp.asarray(_PC)
    vs = jnp.asarray(_VS)
    x_i = jnp.floor((lidar[..., 0] - pc[0]) / vs[0]).astype(jnp.int32)
    y_i = jnp.floor((lidar[..., 1] - pc[1]) / vs[1]).astype(jnp.int32)
    z_ok = (lidar[..., 2] >= pc[2]) & (lidar[..., 2] < pc[5])
    N = Hf * Wf
    op_b = opacity.reshape(B, N)
    base_valid = (op_b >= _MIN_OP) & z_ok
    base_w = op_b * base_valid.astype(jnp.float32)
    ok = base_valid & (x_i >= -1) & (x_i <= _NX) & (y_i >= -1) & (y_i <= _NY)
    wv = jnp.where(ok, base_w, 0.0)
    px = jnp.where(ok, x_i + 1, 0).astype(jnp.int32)
    py = jnp.where(ok, y_i + 1, 0).astype(jnp.int32)

    fh_nhwc = feats.transpose(0, 2, 3, 1)
    xp = jnp.pad(fh_nhwc, ((0, 0), (1, 1), (1, 1), (0, 0)))
    w1 = p['fh_w1'].transpose(2, 3, 1, 0).reshape(9, C, C)
    s1 = (p['fh_g1'] / jnp.sqrt(p['fh_v1'] + 1e-5)).reshape(1, C)
    t1 = (p['fh_b1'] - p['fh_m1'] * s1[0]).reshape(1, C)
    w2 = p['fh_w2'][:, :, 0, 0].T
    b2 = p['fh_bias2'].reshape(1, C)
    pf = _feature_head(xp, w1, s1, t1, w2, b2, wv.reshape(B, N, 1))

    bev = _bev_scatter(py, px, wv, pf, B, N)
    return bev.transpose(0, 3, 1, 2)


# speedup vs baseline: 1.0494x; 1.0013x over previous
import functools

import jax
import jax.numpy as jnp
import numpy as np
from jax.experimental import pallas as pl
from jax.experimental.pallas import tpu as pltpu

_IMG_H = 512
_IMG_W = 512
_C_OUT = 128
_NY = 200
_NX = 200
_DB = 64
_DEPTH_MIN = 1.0
_DEPTH_MAX = 60.0
_SIGMA = 0.8
_MIN_OP = 0.05
_EPS = 1e-6
_PC = np.array([-51.2, -51.2, -20.0, 51.2, 51.2, 20.0], dtype=np.float32)
_VS = np.array([0.512, 0.512, 40.0], dtype=np.float32)
_PAD = 2
_NYP = _NY + 2 * _PAD
_NXP = _NX + 2 * _PAD
_TILES = 8
_ROWS = _NY // _TILES


def _conv(x, w, stride, pad):
    return jax.lax.conv_general_dilated(
        x, w, (stride, stride), [(pad, pad), (pad, pad)],
        dimension_numbers=('NCHW', 'OIHW', 'NCHW'))


def _bn(x, g, b, m, v):
    return (x - m[None, :, None, None]) / jnp.sqrt(v[None, :, None, None] + 1e-5) * g[None, :, None, None] + b[None, :, None, None]


def _fhead_kernel(xp_ref, w1_ref, s_ref, t_ref, w2_ref, b2_ref, wv_ref, o_ref):
    hf = xp_ref.shape[1] - 2
    wf = xp_ref.shape[2] - 2
    c = xp_ref.shape[3]
    acc = jnp.zeros((hf * wf, c), jnp.float32)
    k = 0
    for dy in range(3):
        for dx in range(3):
            xs = xp_ref[0, dy:dy + hf, dx:dx + wf, :].reshape(hf * wf, c)
            acc = acc + jnp.dot(xs, w1_ref[k], preferred_element_type=jnp.float32,
                                precision=jax.lax.Precision.HIGHEST)
            k += 1
    f1 = jnp.maximum(acc * s_ref[...] + t_ref[...], 0.0)
    pf = (jnp.dot(f1, w2_ref[...], preferred_element_type=jnp.float32,
                  precision=jax.lax.Precision.HIGHEST) + b2_ref[...]) * wv_ref[0]
    o_ref[0] = pf


def _feature_head(xp, w1, s, t, w2, b2, wv):
    b, hp, wp, c = xp.shape
    n = (hp - 2) * (wp - 2)
    return pl.pallas_call(
        _fhead_kernel,
        grid=(b,),
        in_specs=[
            pl.BlockSpec((1, hp, wp, c), lambda i: (i, 0, 0, 0)),
            pl.BlockSpec((9, c, c), lambda i: (0, 0, 0)),
            pl.BlockSpec((1, c), lambda i: (0, 0)),
            pl.BlockSpec((1, c), lambda i: (0, 0)),
            pl.BlockSpec((c, c), lambda i: (0, 0)),
            pl.BlockSpec((1, c), lambda i: (0, 0)),
            pl.BlockSpec((1, n, 1), lambda i: (i, 0, 0)),
        ],
        out_specs=pl.BlockSpec((1, n, c), lambda i: (i, 0, 0)),
        out_shape=jax.ShapeDtypeStruct((b, n, c), jnp.float32),
        compiler_params=pltpu.CompilerParams(dimension_semantics=("parallel",)),
    )(xp, w1, s, t, w2, b2, wv)


def _scatter_kernel(py_ref, px_ref, wv_ref, pf_ref, out_ref, canvas_ref, wacc_ref):
    t = pl.program_id(1)
    dyg = jax.lax.broadcasted_iota(jnp.int32, (3, 3), 0).astype(jnp.float32) - 1.0
    dxg = jax.lax.broadcasted_iota(jnp.int32, (3, 3), 1).astype(jnp.float32) - 1.0
    kw = jnp.exp(-(dxg * dxg + dyg * dyg) / (2.0 * _SIGMA * _SIGMA))

    @pl.when(t == 0)
    def _():
        canvas_ref[...] = jnp.zeros(canvas_ref.shape, canvas_ref.dtype)
        wacc_ref[...] = jnp.zeros(wacc_ref.shape, wacc_ref.dtype)

        def body(j, carry):
            w = wv_ref[0, 0, j]
            py = py_ref[0, 0, j]
            px = px_ref[0, 0, j]
            f = pf_ref[0, j, :]
            kww = kw * w
            patch = canvas_ref[pl.ds(py, 3), pl.ds(px, 3), :]
            canvas_ref[pl.ds(py, 3), pl.ds(px, 3), :] = patch + kw[:, :, None] * f[None, None, :]
            wp = wacc_ref[pl.ds(py, 3), pl.ds(px, 3), :]
            wacc_ref[pl.ds(py, 3), pl.ds(px, 3), :] = wp + kww[:, :, None]
            return carry

        jax.lax.fori_loop(0, py_ref.shape[2], body, 0)

    y0 = t * _ROWS + _PAD
    c = canvas_ref[pl.ds(y0, _ROWS), _PAD:_PAD + _NX, :]
    wv = wacc_ref[pl.ds(y0, _ROWS), _PAD:_PAD + _NX, 0:1]
    out_ref[0] = c * (1.0 / jnp.maximum(wv, _EPS))


def _bev_scatter(py, px, wv, pf, B, N):
    return pl.pallas_call(
        _scatter_kernel,
        grid=(B, _TILES),
        in_specs=[
            pl.BlockSpec((1, 1, N), lambda b, t: (b, 0, 0), memory_space=pltpu.SMEM),
            pl.BlockSpec((1, 1, N), lambda b, t: (b, 0, 0), memory_space=pltpu.SMEM),
            pl.BlockSpec((1, 1, N), lambda b, t: (b, 0, 0), memory_space=pltpu.SMEM),
            pl.BlockSpec((1, N, _C_OUT), lambda b, t: (b, 0, 0)),
        ],
        out_specs=pl.BlockSpec((1, _ROWS, _NX, _C_OUT), lambda b, t: (b, t, 0, 0)),
        out_shape=jax.ShapeDtypeStruct((B, _NY, _NX, _C_OUT), jnp.float32),
        scratch_shapes=[
            pltpu.VMEM((_NYP, _NXP, _C_OUT), jnp.float32),
            pltpu.VMEM((_NYP, _NXP, 8), jnp.float32),
        ],
        compiler_params=pltpu.CompilerParams(
            dimension_semantics=("parallel", "arbitrary")),
    )(py.reshape(B, 1, N), px.reshape(B, 1, N), wv.reshape(B, 1, N), pf)


def kernel(images, camera_projection, t_lidar_camera, params):
    p = params
    x = images
    for i in range(1, 5):
        x = _conv(x, p['enc_w%d' % i], 2, 1)
        x = _bn(x, p['enc_g%d' % i], p['enc_b%d' % i], p['enc_m%d' % i], p['enc_v%d' % i])
        x = jnp.maximum(x, 0.0)
    feats = x
    depth_logits = _conv(feats, p['dh_w'], 1, 0) + p['dh_bias'][None, :, None, None]
    depth_probs = jax.nn.softmax(depth_logits, axis=1)
    opacity = jax.nn.sigmoid(_conv(feats, p['oh_w'], 1, 0) + p['oh_bias'][None, :, None, None])[:, 0]
    B, C, Hf, Wf = feats.shape
    depth_values = jnp.linspace(_DEPTH_MIN, _DEPTH_MAX, _DB).astype(jnp.float32)
    z = jnp.sum(depth_probs * depth_values[None, :, None, None], axis=1)[:, None]
    ys = (jnp.arange(Hf, dtype=jnp.float32) + 0.5) * (float(_IMG_H) / float(Hf))
    xs = (jnp.arange(Wf, dtype=jnp.float32) + 0.5) * (float(_IMG_W) / float(Wf))
    yy, xx = jnp.meshgrid(ys, xs, indexing='ij')
    yy = yy[None, None]
    xx = xx[None, None]
    fx = jnp.clip(camera_projection[:, 0, 0], _EPS, None).reshape(B, 1, 1, 1)
    fy = jnp.clip(camera_projection[:, 1, 1], _EPS, None).reshape(B, 1, 1, 1)
    cx = camera_projection[:, 0, 2].reshape(B, 1, 1, 1)
    cy = camera_projection[:, 1, 2].reshape(B, 1, 1, 1)
    x_cam = (xx - cx) * z / fx
    y_cam = (yy - cy) * z / fy
    cam_homo = jnp.stack([x_cam, y_cam, z, jnp.ones_like(z)], axis=-1).reshape(B, -1, 4)
    lidar = jnp.einsum('bij,bnj->bni', t_lidar_camera, cam_homo)[..., :3]
    pc = jnp.asarray(_PC)
    vs = jnp.asarray(_VS)
    x_i = jnp.floor((lidar[..., 0] - pc[0]) / vs[0]).astype(jnp.int32)
    y_i = jnp.floor((lidar[..., 1] - pc[1]) / vs[1]).astype(jnp.int32)
    z_ok = (lidar[..., 2] >= pc[2]) & (lidar[..., 2] < pc[5])
    N = Hf * Wf
    op_b = opacity.reshape(B, N)
    base_valid = (op_b >= _MIN_OP) & z_ok
    base_w = op_b * base_valid.astype(jnp.float32)
    ok = base_valid & (x_i >= -1) & (x_i <= _NX) & (y_i >= -1) & (y_i <= _NY)
    wv = jnp.where(ok, base_w, 0.0)
    px = jnp.where(ok, x_i + 1, 0).astype(jnp.int32)
    py = jnp.where(ok, y_i + 1, 0).astype(jnp.int32)

    fh_nhwc = feats.transpose(0, 2, 3, 1)
    xp = jnp.pad(fh_nhwc, ((0, 0), (1, 1), (1, 1), (0, 0)))
    w1 = p['fh_w1'].transpose(2, 3, 1, 0).reshape(9, C, C)
    s1 = (p['fh_g1'] / jnp.sqrt(p['fh_v1'] + 1e-5)).reshape(1, C)
    t1 = (p['fh_b1'] - p['fh_m1'] * s1[0]).reshape(1, C)
    w2 = p['fh_w2'][:, :, 0, 0].T
    b2 = p['fh_bias2'].reshape(1, C)
    pf = _feature_head(xp, w1, s1, t1, w2, b2, wv.reshape(B, N, 1))

    bev = _bev_scatter(py, px, wv, pf, B, N)
    return bev.transpose(0, 3, 1, 2)
